# baseline (speedup 1.0000x reference)
import functools

import jax
import jax.numpy as jnp
from jax import lax
from jax.experimental import pallas as pl
from jax.experimental.pallas import tpu as pltpu
from jax.experimental.pallas import tpu_sc as plsc

B, N, S, D = 4096, 10, 20, 32
VOCAB_ROWS = 1000000
R = B * S
NUM_CORES = 2
NUM_SUBCORES = 16
NW = NUM_CORES * NUM_SUBCORES
BPW = B // NW
G = 8
C = G * S
NCHUNK = BPW // G
LANES = 16


def _cbow_body(idx_hbm, table_hbm, out_hbm, idx_v, acc_v, sem):
    wid = lax.axis_index("s") * NUM_CORES + lax.axis_index("c")
    bbase = wid * BPW

    def chunk_body(ci, carry):
        b0 = bbase + ci * G
        pltpu.sync_copy(idx_hbm.at[pl.ds(b0, G)], idx_v)
        first = [
            pltpu.async_copy(
                table_hbm.at[idx_v.at[g, 0]], acc_v.at[g, 0], sem)
            for g in range(G)
        ]
        for cp in first:
            cp.wait()
        rest = [
            pltpu.async_copy(
                table_hbm.at[idx_v.at[g, n]], acc_v.at[g, 0], sem,
                add=True)
            for g in range(G)
            for n in range(1, N)
        ]
        for cp in rest:
            cp.wait()

        def row_body(r, c2):
            g = r // S
            s = r % S
            for h in range(0, D, LANES):
                acc_v[g, 0, s, pl.ds(h, LANES)] = (
                    acc_v[g, 0, s, pl.ds(h, LANES)] * 0.1)
            return c2

        lax.fori_loop(0, C, row_body, 0, unroll=4)
        pltpu.sync_copy(acc_v, out_hbm.at[pl.ds(b0, G)])
        return carry

    lax.fori_loop(0, NCHUNK, chunk_body, 0)


@jax.jit
def kernel(x, table):
    mesh = plsc.VectorSubcoreMesh(core_axis_name="c", subcore_axis_name="s")
    run = pl.kernel(
        _cbow_body,
        mesh=mesh,
        out_type=jax.ShapeDtypeStruct((B, 1, S, D), jnp.float32),
        scratch_types=[
            pltpu.VMEM((G, N, S), jnp.int32),
            pltpu.VMEM((G, 1, S, D), jnp.float32),
            pltpu.SemaphoreType.DMA,
        ],
        compiler_params=pltpu.CompilerParams(use_tc_tiling_on_sc=False),
    )
    return run(x.astype(jnp.int32), table)


# speedup vs baseline: 1.0156x; 1.0156x over previous
import jax
import jax.numpy as jnp
from jax import lax
from jax.experimental import pallas as pl
from jax.experimental.pallas import tpu as pltpu
from jax.experimental.pallas import tpu_sc as plsc

B, N, S, D = 4096, 10, 20, 32
VOCAB_ROWS = 1000000
R = B * S
NUM_CORES = 2
NUM_SUBCORES = 16
NW = NUM_CORES * NUM_SUBCORES
BPW = B // NW
G = 8
C = G * S
NCHUNK = BPW // G
LANES = 16


def _cbow_body(idx_hbm, table_hbm, out_hbm, idx_v, acc_v, sem):
    wid = lax.axis_index("s") * NUM_CORES + lax.axis_index("c")
    bbase = wid * BPW

    def chunk_body(ci, carry):
        b0 = bbase + ci * G
        pltpu.sync_copy(idx_hbm.at[pl.ds(b0, G)], idx_v)
        first = [
            pltpu.async_copy(
                table_hbm.at[idx_v.at[g, 0]], acc_v.at[pl.ds(g * S, S)], sem)
            for g in range(G)
        ]
        for cp in first:
            cp.wait()
        rest = [
            pltpu.async_copy(
                table_hbm.at[idx_v.at[g, n]], acc_v.at[pl.ds(g * S, S)], sem,
                add=True)
            for g in range(G)
            for n in range(1, N)
        ]
        for cp in rest:
            cp.wait()

        def row_body(r, c2):
            for h in range(0, D, LANES):
                acc_v[r, pl.ds(h, LANES)] = acc_v[r, pl.ds(h, LANES)] * 0.1
            return c2

        lax.fori_loop(0, C, row_body, 0, unroll=4)
        pltpu.sync_copy(acc_v, out_hbm.at[pl.ds(b0 * S, C)])
        return carry

    lax.fori_loop(0, NCHUNK, chunk_body, 0)


@jax.jit
def kernel(x, table):
    mesh = plsc.VectorSubcoreMesh(core_axis_name="c", subcore_axis_name="s")
    run = pl.kernel(
        _cbow_body,
        mesh=mesh,
        out_type=jax.ShapeDtypeStruct((R, D), jnp.float32),
        scratch_types=[
            pltpu.VMEM((G, N, S), jnp.int32),
            pltpu.VMEM((C, D), jnp.float32),
            pltpu.SemaphoreType.DMA,
        ],
        compiler_params=pltpu.CompilerParams(use_tc_tiling_on_sc=False),
    )
    out = run(x.astype(jnp.int32), table)
    return out.reshape(B, 1, S, D)
